# baseline (speedup 1.0000x reference)
import functools
import itertools

import jax
import jax.numpy as jnp
import numpy as np
from jax import lax
from jax.experimental import pallas as pl
from jax.experimental.pallas import tpu as pltpu
from jax.experimental.pallas import tpu_sc as plsc

B, N, K_NN, M = 16, 2048, 16, 2025
MPAD = 2048
NEG_INF = float("-inf")


def _knn_cov_body(pts_ref, ptsT_ref, idx_ref, cov_ref):
    b = pl.program_id(0)
    xb = pts_ref[0]
    pT = ptsT_ref[0]
    col = lax.broadcasted_iota(jnp.int32, (1, N), 1)

    inner = -2.0 * jnp.dot(xb, pT, preferred_element_type=jnp.float32)
    xxb = xb[:, 0:1] ** 2 + xb[:, 1:2] ** 2 + xb[:, 2:3] ** 2
    xxa = pT[0:1, :] ** 2 + pT[1:2, :] ** 2 + pT[2:3, :] ** 2
    npw = jnp.maximum((xxb + inner) + xxa, 1.17549435e-38)
    ub = lax.bitcast_convert_type(npw, jnp.int32)
    fenc = lax.bitcast_convert_type((ub & ~2047) | col, jnp.float32)

    idx_cols = []
    a0 = a1 = None
    for k in range(K_NN):
        m = jnp.min(fenc, axis=1, keepdims=True)
        a = lax.bitcast_convert_type(m, jnp.int32) & 2047
        idx_cols.append(a)
        if k == 0:
            a0 = a
        elif k == 1:
            a1 = a
        if k < K_NN - 1:
            fenc = jnp.where(fenc == m, jnp.inf, fenc)

    idx = jnp.concatenate(idx_cols, axis=1)
    idx_ref[0] = idx + b * N

    colf = col.astype(jnp.float32)
    npw1 = jnp.where(col == a0, jnp.inf, npw)
    m1 = jnp.min(npw1, axis=1, keepdims=True)
    a1 = jnp.min(jnp.where(npw1 == m1, colf, jnp.float32(N)),
                 axis=1, keepdims=True).astype(jnp.int32)

    oh0 = (col == a0).astype(jnp.float32)
    oh1 = (col == a1).astype(jnp.float32)
    nbr = jnp.dot(jnp.concatenate([oh0, oh1], axis=0), pT.T,
                  preferred_element_type=jnp.float32)
    rb = xb.shape[0]
    n0 = [nbr[:rb, d:d + 1] for d in range(3)]
    n1 = [nbr[rb:, d:d + 1] for d in range(3)]
    outer = [n0[u] * n1[v] for u in range(3) for v in range(3)]
    cov_ref[0] = jnp.concatenate([xb] + outer, axis=1)


def _knn_cov(x, rb=256):
    nb = x.shape[0]
    xT = x.transpose(0, 2, 1)
    grid = (nb, N // rb)
    return pl.pallas_call(
        _knn_cov_body,
        grid=grid,
        in_specs=[
            pl.BlockSpec((1, rb, 3), lambda b, r: (b, r, 0)),
            pl.BlockSpec((1, 3, N), lambda b, r: (b, 0, 0)),
        ],
        out_specs=[
            pl.BlockSpec((1, rb, K_NN), lambda b, r: (b, r, 0)),
            pl.BlockSpec((1, rb, 12), lambda b, r: (b, r, 0)),
        ],
        out_shape=[
            jax.ShapeDtypeStruct((nb, N, K_NN), jnp.int32),
            jax.ShapeDtypeStruct((nb, N, 12), jnp.float32),
        ],
    )(x, xT)


def _enc_body(x_ref, w0_ref, b0_ref, w1_ref, b1_ref, w2_ref, b2_ref, o_ref):
    h = jax.nn.relu(jnp.dot(x_ref[...], w0_ref[...],
                            preferred_element_type=jnp.float32) + b0_ref[...])
    h = jax.nn.relu(jnp.dot(h, w1_ref[...],
                            preferred_element_type=jnp.float32) + b1_ref[...])
    h = jax.nn.relu(jnp.dot(h, w2_ref[...],
                            preferred_element_type=jnp.float32) + b2_ref[...])
    o_ref[...] = h


def _encoder(cov, w0, b0, w1, b1, w2, b2, rb=2048):
    R = cov.shape[0]
    full = lambda a: pl.BlockSpec(a.shape, lambda i: (0,) * a.ndim)
    args = (w0.T, b0.reshape(1, -1), w1.T, b1.reshape(1, -1), w2.T,
            b2.reshape(1, -1))
    return pl.pallas_call(
        _enc_body,
        grid=(R // rb,),
        in_specs=[pl.BlockSpec((rb, 12), lambda i: (i, 0))] +
                 [full(a) for a in args],
        out_specs=pl.BlockSpec((rb, 64), lambda i: (i, 0)),
        out_shape=jax.ShapeDtypeStruct((R, 64), jnp.float32),
    )(cov, *args)


def _lin_conv_body(x_ref, wl_ref, bl_ref, wc_ref, bc_ref, o_ref):
    h = jnp.dot(x_ref[...], wl_ref[...],
                preferred_element_type=jnp.float32) + bl_ref[...]
    h = jax.nn.relu(jnp.dot(h, wc_ref[...],
                            preferred_element_type=jnp.float32) + bc_ref[...])
    o_ref[...] = h


def _lin_conv(x, wl, bl, wc, bc, dout, rb=2048):
    R = x.shape[0]
    full = lambda a: pl.BlockSpec(a.shape, lambda i: (0,) * a.ndim)
    args = (wl.T, bl.reshape(1, -1), wc.T, bc.reshape(1, -1))
    return pl.pallas_call(
        _lin_conv_body,
        grid=(R // rb,),
        in_specs=[pl.BlockSpec((rb, x.shape[1]), lambda i: (i, 0))] +
                 [full(a) for a in args],
        out_specs=pl.BlockSpec((rb, dout), lambda i: (i, 0)),
        out_shape=jax.ShapeDtypeStruct((R, dout), jnp.float32),
    )(x, *args)


def _globalfeat_body(x_ref, wl_ref, bl_ref, wc_ref, bc_ref, o_ref):
    h = jnp.dot(x_ref[0], wl_ref[...],
                preferred_element_type=jnp.float32) + bl_ref[...]
    h = jnp.dot(h, wc_ref[...],
                preferred_element_type=jnp.float32) + bc_ref[...]
    o_ref[0] = jnp.max(h, axis=0, keepdims=True)


def _globalfeat(x, wl, bl, wc, bc):
    nb = x.shape[0]
    full = lambda a: pl.BlockSpec(a.shape, lambda b: (0,) * a.ndim)
    args = (wl.T, bl.reshape(1, -1), wc.T, bc.reshape(1, -1))
    return pl.pallas_call(
        _globalfeat_body,
        grid=(nb,),
        in_specs=[pl.BlockSpec((1, N, 128), lambda b: (b, 0, 0))] +
                 [full(a) for a in args],
        out_specs=pl.BlockSpec((1, 1, 1024), lambda b: (b, 0, 0)),
        out_shape=jax.ShapeDtypeStruct((nb, 1, 1024), jnp.float32),
    )(x, *args)


def _mlp2_body(x_ref, w0_ref, b0_ref, w1_ref, b1_ref, o_ref):
    h = jax.nn.relu(jnp.dot(x_ref[...], w0_ref[...],
                            preferred_element_type=jnp.float32) + b0_ref[...])
    o_ref[...] = jnp.dot(h, w1_ref[...],
                         preferred_element_type=jnp.float32) + b1_ref[...]


def _mlp2(x, w0, b0, w1, b1):
    full = lambda a: pl.BlockSpec(a.shape, lambda: (0,) * a.ndim)
    args = (w0.T, b0.reshape(1, -1), w1.T, b1.reshape(1, -1))
    return pl.pallas_call(
        _mlp2_body,
        in_specs=[pl.BlockSpec(x.shape, lambda: (0, 0))] +
                 [full(a) for a in args],
        out_specs=pl.BlockSpec((B, 512), lambda: (0, 0)),
        out_shape=jax.ShapeDtypeStruct((B, 512), jnp.float32),
    )(x, *args)


def _decoder_body(code_ref, grid_ref,
                  a1_ref, g1_ref, b10_ref, w11_ref, b11_ref, w12_ref, b12_ref,
                  a2_ref, g2_ref, b20_ref, w21_ref, b21_ref, w22_ref, b22_ref,
                  o_ref):
    code = code_ref[0]
    gp = grid_ref[...]
    ct1 = jnp.dot(code, a1_ref[...],
                  preferred_element_type=jnp.float32) + b10_ref[...]
    gt1 = (gp[:, 0:1] * g1_ref[0:1, :] + gp[:, 1:2] * g1_ref[1:2, :])
    h = jax.nn.relu(ct1 + gt1)
    h = jax.nn.relu(jnp.dot(h, w11_ref[...],
                            preferred_element_type=jnp.float32) + b11_ref[...])
    f1 = jnp.dot(h, w12_ref[...],
                 preferred_element_type=jnp.float32) + b12_ref[...]
    ct2 = jnp.dot(code, a2_ref[...],
                  preferred_element_type=jnp.float32) + b20_ref[...]
    h = jax.nn.relu(ct2 + jnp.dot(f1, g2_ref[...],
                                  preferred_element_type=jnp.float32))
    h = jax.nn.relu(jnp.dot(h, w21_ref[...],
                            preferred_element_type=jnp.float32) + b21_ref[...])
    o_ref[0] = jnp.dot(h, w22_ref[...],
                       preferred_element_type=jnp.float32) + b22_ref[...]


def _decoder(code, gridp, f1_w0, f1_b0, f1_w1, f1_b1, f1_w2, f1_b2,
             f2_w0, f2_b0, f2_w1, f2_b1, f2_w2, f2_b2, mb=1024):
    full = lambda a: pl.BlockSpec(a.shape, lambda b, m: (0,) * a.ndim)
    args = (f1_w0[:, :512].T, f1_w0[:, 512:].T, f1_b0.reshape(1, -1),
            f1_w1.T, f1_b1.reshape(1, -1), f1_w2.T, f1_b2.reshape(1, -1),
            f2_w0[:, :512].T, f2_w0[:, 512:].T, f2_b0.reshape(1, -1),
            f2_w1.T, f2_b1.reshape(1, -1), f2_w2.T, f2_b2.reshape(1, -1))
    code = code.reshape(B, 1, 512)
    return pl.pallas_call(
        _decoder_body,
        grid=(B, MPAD // mb),
        in_specs=[pl.BlockSpec((1, 1, 512), lambda b, m: (b, 0, 0)),
                  pl.BlockSpec((mb, 2), lambda b, m: (m, 0))] +
                 [full(a) for a in args],
        out_specs=pl.BlockSpec((1, mb, 3), lambda b, m: (b, m, 0)),
        out_shape=jax.ShapeDtypeStruct((B, MPAD, 3), jnp.float32),
    )(code, gridp, *args)


def _sc_maxpool(table, idx_flat, D, c_pts):
    R = table.shape[0]
    NW = 32
    per_w = R // NW
    rows = c_pts * K_NN
    n_chunks = per_w // c_pts
    mesh = plsc.VectorSubcoreMesh(core_axis_name="c", subcore_axis_name="s")

    @functools.partial(
        pl.kernel, mesh=mesh,
        compiler_params=pltpu.CompilerParams(use_tc_tiling_on_sc=False),
        out_type=jax.ShapeDtypeStruct((R, D), jnp.float32),
        scratch_types=[
            pltpu.VMEM((rows,), jnp.int32),
            pltpu.VMEM((rows, D), jnp.float32),
            pltpu.VMEM((c_pts, D), jnp.float32),
            pltpu.SemaphoreType.DMA,
        ],
    )
    def mp(table_hbm, idx_hbm, out_hbm, idx_v, rows_v, out_v, sem):
        wid = lax.axis_index("s") * 2 + lax.axis_index("c")

        def chunk_body(i, carry):
            base_pt = wid * per_w + i * c_pts
            pltpu.sync_copy(idx_hbm.at[pl.ds(base_pt * K_NN, rows)], idx_v)
            handles = [
                pltpu.async_copy(
                    table_hbm.at[idx_v.at[pl.ds(g * 128, 128)]],
                    rows_v.at[pl.ds(g * 128, 128)],
                    sem)
                for g in range(rows // 128)
            ]
            for h in handles:
                h.wait()

            def pt_body(p, c):
                for cc in range(D // 16):
                    acc = rows_v[p * K_NN, pl.ds(cc * 16, 16)]
                    for n in range(1, K_NN):
                        acc = jnp.maximum(
                            acc, rows_v[p * K_NN + n, pl.ds(cc * 16, 16)])
                    out_v[p, pl.ds(cc * 16, 16)] = acc
                return c

            lax.fori_loop(0, c_pts, pt_body, 0)
            pltpu.sync_copy(out_v, out_hbm.at[pl.ds(base_pt, c_pts)])
            return carry

        lax.fori_loop(0, n_chunks, chunk_body, 0)

    return mp(table, idx_flat)


def _make_grid():
    g = np.linspace(-0.3, 0.3, 45)
    pts = np.array(list(itertools.product(g, g)), dtype=np.float32)
    gp = np.zeros((MPAD, 2), dtype=np.float32)
    gp[:M] = pts
    return jnp.asarray(gp)


def kernel(x, enc_c0_w, enc_c0_b, enc_c1_w, enc_c1_b, enc_c2_w, enc_c2_b,
           lin1_w, lin1_b, conv1_w, conv1_b, lin2_w, lin2_b,
           conv2_w, conv2_b, mlp2_w0, mlp2_b0, mlp2_w1, mlp2_b1,
           f1_w0, f1_b0, f1_w1, f1_b1, f1_w2, f1_b2,
           f2_w0, f2_b0, f2_w1, f2_b1, f2_w2, f2_b2):
    gfs = []
    hb = B // 2
    for h in range(2):
        xh = x[h * hb:(h + 1) * hb]
        idx, cov = _knn_cov(xh)
        idx_flat = idx.reshape(-1)
        cov2d = cov.reshape(hb * N, 12)
        h1 = _encoder(cov2d, enc_c0_w, enc_c0_b, enc_c1_w, enc_c1_b,
                      enc_c2_w, enc_c2_b)
        h1p = _sc_maxpool(h1, idx_flat, 64, 64)
        h2 = _lin_conv(h1p, lin1_w, lin1_b, conv1_w, conv1_b, 128)
        h2p = _sc_maxpool(h2, idx_flat, 128, 32)
        gfs.append(_globalfeat(h2p.reshape(hb, N, 128),
                               lin2_w, lin2_b, conv2_w, conv2_b))
    gf = jnp.concatenate(gfs, axis=0)
    code = _mlp2(gf.reshape(B, 1024), mlp2_w0, mlp2_b0, mlp2_w1, mlp2_b1)
    out = _decoder(code, _make_grid(), f1_w0, f1_b0, f1_w1, f1_b1,
                   f1_w2, f1_b2, f2_w0, f2_b0, f2_w1, f2_b1, f2_w2, f2_b2)
    return out[:, :M, :]


# speedup vs baseline: 1.0707x; 1.0707x over previous
import functools
import itertools

import jax
import jax.numpy as jnp
import numpy as np
from jax import lax
from jax.experimental import pallas as pl
from jax.experimental.pallas import tpu as pltpu
from jax.experimental.pallas import tpu_sc as plsc

B, N, K_NN, M = 16, 2048, 16, 2025
MPAD = 2048
NEG_INF = float("-inf")


def _knn_cov_body(pts_ref, ptsT_ref, idx_ref, cov_ref):
    b = pl.program_id(0)
    xb = pts_ref[0]
    pT = ptsT_ref[0]
    col = lax.broadcasted_iota(jnp.int32, (1, N), 1)

    inner = -2.0 * jnp.dot(xb, pT, preferred_element_type=jnp.float32)
    xxb = xb[:, 0:1] ** 2 + xb[:, 1:2] ** 2 + xb[:, 2:3] ** 2
    xxa = pT[0:1, :] ** 2 + pT[1:2, :] ** 2 + pT[2:3, :] ** 2
    npw = jnp.maximum((xxb + inner) + xxa, 1.17549435e-38)
    ub = lax.bitcast_convert_type(npw, jnp.int32)
    fenc = lax.bitcast_convert_type((ub & ~2047) | col, jnp.float32)

    idx_cols = []
    a0 = a1 = None
    for k in range(K_NN):
        m = jnp.min(fenc, axis=1, keepdims=True)
        a = lax.bitcast_convert_type(m, jnp.int32) & 2047
        idx_cols.append(a)
        if k == 0:
            a0 = a
        elif k == 1:
            a1 = a
        if k < K_NN - 1:
            fenc = jnp.where(fenc == m, jnp.inf, fenc)

    idx = jnp.concatenate(idx_cols, axis=1)
    idx_ref[0] = idx + b * N

    colf = col.astype(jnp.float32)
    npw1 = jnp.where(col == a0, jnp.inf, npw)
    m1 = jnp.min(npw1, axis=1, keepdims=True)
    a1 = jnp.min(jnp.where(npw1 == m1, colf, jnp.float32(N)),
                 axis=1, keepdims=True).astype(jnp.int32)

    oh0 = (col == a0).astype(jnp.float32)
    oh1 = (col == a1).astype(jnp.float32)
    nbr = jnp.dot(jnp.concatenate([oh0, oh1], axis=0), pT.T,
                  preferred_element_type=jnp.float32)
    rb = xb.shape[0]
    n0 = [nbr[:rb, d:d + 1] for d in range(3)]
    n1 = [nbr[rb:, d:d + 1] for d in range(3)]
    outer = [n0[u] * n1[v] for u in range(3) for v in range(3)]
    cov_ref[0] = jnp.concatenate([xb] + outer, axis=1)


def _knn_cov(x, rb=256):
    nb = x.shape[0]
    xT = x.transpose(0, 2, 1)
    grid = (nb, N // rb)
    return pl.pallas_call(
        _knn_cov_body,
        grid=grid,
        in_specs=[
            pl.BlockSpec((1, rb, 3), lambda b, r: (b, r, 0)),
            pl.BlockSpec((1, 3, N), lambda b, r: (b, 0, 0)),
        ],
        out_specs=[
            pl.BlockSpec((1, rb, K_NN), lambda b, r: (b, r, 0)),
            pl.BlockSpec((1, rb, 12), lambda b, r: (b, r, 0)),
        ],
        out_shape=[
            jax.ShapeDtypeStruct((nb, N, K_NN), jnp.int32),
            jax.ShapeDtypeStruct((nb, N, 12), jnp.float32),
        ],
    )(x, xT)


def _enc_body(x_ref, w0_ref, b0_ref, w1_ref, b1_ref, w2_ref, b2_ref, o_ref):
    h = jax.nn.relu(jnp.dot(x_ref[...], w0_ref[...],
                            preferred_element_type=jnp.float32) + b0_ref[...])
    h = jax.nn.relu(jnp.dot(h, w1_ref[...],
                            preferred_element_type=jnp.float32) + b1_ref[...])
    h = jax.nn.relu(jnp.dot(h, w2_ref[...],
                            preferred_element_type=jnp.float32) + b2_ref[...])
    o_ref[...] = h


def _encoder(cov, w0, b0, w1, b1, w2, b2, rb=2048):
    R = cov.shape[0]
    full = lambda a: pl.BlockSpec(a.shape, lambda i: (0,) * a.ndim)
    args = (w0.T, b0.reshape(1, -1), w1.T, b1.reshape(1, -1), w2.T,
            b2.reshape(1, -1))
    return pl.pallas_call(
        _enc_body,
        grid=(R // rb,),
        in_specs=[pl.BlockSpec((rb, 12), lambda i: (i, 0))] +
                 [full(a) for a in args],
        out_specs=pl.BlockSpec((rb, 64), lambda i: (i, 0)),
        out_shape=jax.ShapeDtypeStruct((R, 64), jnp.float32),
    )(cov, *args)


def _lin_conv_body(x_ref, wl_ref, bl_ref, wc_ref, bc_ref, o_ref):
    h = jnp.dot(x_ref[...], wl_ref[...],
                preferred_element_type=jnp.float32) + bl_ref[...]
    h = jax.nn.relu(jnp.dot(h, wc_ref[...],
                            preferred_element_type=jnp.float32) + bc_ref[...])
    o_ref[...] = h


def _lin_conv(x, wl, bl, wc, bc, dout, rb=2048):
    R = x.shape[0]
    full = lambda a: pl.BlockSpec(a.shape, lambda i: (0,) * a.ndim)
    args = (wl.T, bl.reshape(1, -1), wc.T, bc.reshape(1, -1))
    return pl.pallas_call(
        _lin_conv_body,
        grid=(R // rb,),
        in_specs=[pl.BlockSpec((rb, x.shape[1]), lambda i: (i, 0))] +
                 [full(a) for a in args],
        out_specs=pl.BlockSpec((rb, dout), lambda i: (i, 0)),
        out_shape=jax.ShapeDtypeStruct((R, dout), jnp.float32),
    )(x, *args)


def _globalfeat_body(x_ref, wl_ref, bl_ref, wc_ref, bc_ref, o_ref):
    h = jnp.dot(x_ref[0], wl_ref[...],
                preferred_element_type=jnp.float32) + bl_ref[...]
    h = jnp.dot(h, wc_ref[...],
                preferred_element_type=jnp.float32) + bc_ref[...]
    o_ref[0] = jnp.max(h, axis=0, keepdims=True)


def _globalfeat(x, wl, bl, wc, bc):
    nb = x.shape[0]
    full = lambda a: pl.BlockSpec(a.shape, lambda b: (0,) * a.ndim)
    args = (wl.T, bl.reshape(1, -1), wc.T, bc.reshape(1, -1))
    return pl.pallas_call(
        _globalfeat_body,
        grid=(nb,),
        in_specs=[pl.BlockSpec((1, N, 128), lambda b: (b, 0, 0))] +
                 [full(a) for a in args],
        out_specs=pl.BlockSpec((1, 1, 1024), lambda b: (b, 0, 0)),
        out_shape=jax.ShapeDtypeStruct((nb, 1, 1024), jnp.float32),
    )(x, *args)


def _mlp2_body(x_ref, w0_ref, b0_ref, w1_ref, b1_ref, o_ref):
    h = jax.nn.relu(jnp.dot(x_ref[...], w0_ref[...],
                            preferred_element_type=jnp.float32) + b0_ref[...])
    o_ref[...] = jnp.dot(h, w1_ref[...],
                         preferred_element_type=jnp.float32) + b1_ref[...]


def _mlp2(x, w0, b0, w1, b1):
    full = lambda a: pl.BlockSpec(a.shape, lambda: (0,) * a.ndim)
    args = (w0.T, b0.reshape(1, -1), w1.T, b1.reshape(1, -1))
    return pl.pallas_call(
        _mlp2_body,
        in_specs=[pl.BlockSpec(x.shape, lambda: (0, 0))] +
                 [full(a) for a in args],
        out_specs=pl.BlockSpec((B, 512), lambda: (0, 0)),
        out_shape=jax.ShapeDtypeStruct((B, 512), jnp.float32),
    )(x, *args)


def _decoder_body(code_ref, grid_ref,
                  a1_ref, g1_ref, b10_ref, w11_ref, b11_ref, w12_ref, b12_ref,
                  a2_ref, g2_ref, b20_ref, w21_ref, b21_ref, w22_ref, b22_ref,
                  o_ref):
    code = code_ref[0]
    gp = grid_ref[...]
    ct1 = jnp.dot(code, a1_ref[...],
                  preferred_element_type=jnp.float32) + b10_ref[...]
    gt1 = (gp[:, 0:1] * g1_ref[0:1, :] + gp[:, 1:2] * g1_ref[1:2, :])
    h = jax.nn.relu(ct1 + gt1)
    h = jax.nn.relu(jnp.dot(h, w11_ref[...],
                            preferred_element_type=jnp.float32) + b11_ref[...])
    f1 = jnp.dot(h, w12_ref[...],
                 preferred_element_type=jnp.float32) + b12_ref[...]
    ct2 = jnp.dot(code, a2_ref[...],
                  preferred_element_type=jnp.float32) + b20_ref[...]
    h = jax.nn.relu(ct2 + jnp.dot(f1, g2_ref[...],
                                  preferred_element_type=jnp.float32))
    h = jax.nn.relu(jnp.dot(h, w21_ref[...],
                            preferred_element_type=jnp.float32) + b21_ref[...])
    o_ref[0] = jnp.dot(h, w22_ref[...],
                       preferred_element_type=jnp.float32) + b22_ref[...]


def _decoder(code, gridp, f1_w0, f1_b0, f1_w1, f1_b1, f1_w2, f1_b2,
             f2_w0, f2_b0, f2_w1, f2_b1, f2_w2, f2_b2, mb=1024):
    full = lambda a: pl.BlockSpec(a.shape, lambda b, m: (0,) * a.ndim)
    args = (f1_w0[:, :512].T, f1_w0[:, 512:].T, f1_b0.reshape(1, -1),
            f1_w1.T, f1_b1.reshape(1, -1), f1_w2.T, f1_b2.reshape(1, -1),
            f2_w0[:, :512].T, f2_w0[:, 512:].T, f2_b0.reshape(1, -1),
            f2_w1.T, f2_b1.reshape(1, -1), f2_w2.T, f2_b2.reshape(1, -1))
    code = code.reshape(B, 1, 512)
    return pl.pallas_call(
        _decoder_body,
        grid=(B, MPAD // mb),
        in_specs=[pl.BlockSpec((1, 1, 512), lambda b, m: (b, 0, 0)),
                  pl.BlockSpec((mb, 2), lambda b, m: (m, 0))] +
                 [full(a) for a in args],
        out_specs=pl.BlockSpec((1, mb, 3), lambda b, m: (b, m, 0)),
        out_shape=jax.ShapeDtypeStruct((B, MPAD, 3), jnp.float32),
    )(code, gridp, *args)


def _sc_maxpool(table, idx_flat, D, c_pts):
    R = table.shape[0]
    NW = 32
    per_w = R // NW
    rows = c_pts * K_NN
    n_chunks = per_w // c_pts
    mesh = plsc.VectorSubcoreMesh(core_axis_name="c", subcore_axis_name="s")

    n_g = rows // 128

    @functools.partial(
        pl.kernel, mesh=mesh,
        compiler_params=pltpu.CompilerParams(use_tc_tiling_on_sc=False),
        out_type=jax.ShapeDtypeStruct((R, D), jnp.float32),
        scratch_types=[
            pltpu.VMEM((2, rows), jnp.int32),
            pltpu.VMEM((2, rows, D), jnp.float32),
            pltpu.VMEM((2, c_pts, D), jnp.float32),
            pltpu.SemaphoreType.DMA,
            pltpu.SemaphoreType.DMA,
        ],
    )
    def mp(table_hbm, idx_hbm, out_hbm, idx_v, rows_v, out_v, sem0, sem1):
        wid = lax.axis_index("s") * 2 + lax.axis_index("c")
        sems = (sem0, sem1)

        def issue(chunk, buf):
            base_pt = wid * per_w + chunk * c_pts
            pltpu.sync_copy(idx_hbm.at[pl.ds(base_pt * K_NN, rows)],
                            idx_v.at[buf])
            for g in range(n_g):
                pltpu.async_copy(
                    table_hbm.at[idx_v.at[buf].at[pl.ds(g * 128, 128)]],
                    rows_v.at[buf].at[pl.ds(g * 128, 128)],
                    sems[buf])

        def drain(buf):
            for g in range(n_g):
                pltpu.make_async_copy(
                    table_hbm.at[pl.ds(0, 128)],
                    rows_v.at[buf].at[pl.ds(g * 128, 128)],
                    sems[buf]).wait()

        issue(0, 0)

        def pair_body(i, carry):
            for par in range(2):
                chunk = 2 * i + par
                drain(par)

                @pl.when(chunk + 1 < n_chunks)
                def _():
                    issue(chunk + 1, 1 - par)

                rv = rows_v.at[par]
                ov = out_v.at[par]

                def pt_body(p, c):
                    for cc in range(D // 16):
                        acc = rv[p * K_NN, pl.ds(cc * 16, 16)]
                        for n in range(1, K_NN):
                            acc = jnp.maximum(
                                acc, rv[p * K_NN + n, pl.ds(cc * 16, 16)])
                        ov[p, pl.ds(cc * 16, 16)] = acc
                    return c

                lax.fori_loop(0, c_pts, pt_body, 0)
                base_pt = wid * per_w + chunk * c_pts
                pltpu.sync_copy(ov, out_hbm.at[pl.ds(base_pt, c_pts)])
            return carry

        lax.fori_loop(0, n_chunks // 2, pair_body, 0)

    return mp(table, idx_flat)


def _make_grid():
    g = np.linspace(-0.3, 0.3, 45)
    pts = np.array(list(itertools.product(g, g)), dtype=np.float32)
    gp = np.zeros((MPAD, 2), dtype=np.float32)
    gp[:M] = pts
    return jnp.asarray(gp)


def kernel(x, enc_c0_w, enc_c0_b, enc_c1_w, enc_c1_b, enc_c2_w, enc_c2_b,
           lin1_w, lin1_b, conv1_w, conv1_b, lin2_w, lin2_b,
           conv2_w, conv2_b, mlp2_w0, mlp2_b0, mlp2_w1, mlp2_b1,
           f1_w0, f1_b0, f1_w1, f1_b1, f1_w2, f1_b2,
           f2_w0, f2_b0, f2_w1, f2_b1, f2_w2, f2_b2):
    gfs = []
    hb = B // 2
    for h in range(2):
        xh = x[h * hb:(h + 1) * hb]
        idx, cov = _knn_cov(xh)
        idx_flat = idx.reshape(-1)
        cov2d = cov.reshape(hb * N, 12)
        h1 = _encoder(cov2d, enc_c0_w, enc_c0_b, enc_c1_w, enc_c1_b,
                      enc_c2_w, enc_c2_b)
        h1p = _sc_maxpool(h1, idx_flat, 64, 32)
        h2 = _lin_conv(h1p, lin1_w, lin1_b, conv1_w, conv1_b, 128)
        h2p = _sc_maxpool(h2, idx_flat, 128, 16)
        gfs.append(_globalfeat(h2p.reshape(hb, N, 128),
                               lin2_w, lin2_b, conv2_w, conv2_b))
    gf = jnp.concatenate(gfs, axis=0)
    code = _mlp2(gf.reshape(B, 1024), mlp2_w0, mlp2_b0, mlp2_w1, mlp2_b1)
    out = _decoder(code, _make_grid(), f1_w0, f1_b0, f1_w1, f1_b1,
                   f1_w2, f1_b2, f2_w0, f2_b0, f2_w1, f2_b1, f2_w2, f2_b2)
    return out[:, :M, :]
